# baseline (speedup 1.0000x reference)
import jax
import jax.numpy as jnp
from jax.experimental import pallas as pl

B, T, D = 16, 2048, 512
G, V = 2, 1024
DG = D // G
N = B * T
BT = 1024


def _vq_kernel(x_ref, w_ref, b_ref, cb_ref, out_ref, probs_ref):
    i = pl.program_id(0)

    @pl.when(i == 0)
    def _init():
        probs_ref[...] = jnp.zeros_like(probs_ref)

    logits = jnp.dot(x_ref[...], w_ref[...], preferred_element_type=jnp.float32)
    logits = logits + b_ref[...]
    ones = jnp.ones((8, BT), jnp.bfloat16)
    for g in range(G):
        lg = logits[:, g * V:(g + 1) * V]
        m = jnp.max(lg, axis=1, keepdims=True)
        oh = (lg == m).astype(jnp.bfloat16)
        out_ref[:, g * DG:(g + 1) * DG] = jnp.dot(
            oh, cb_ref[g * V:(g + 1) * V, :], preferred_element_type=jnp.float32)
        cnt = jnp.dot(ones, oh, preferred_element_type=jnp.float32)
        probs_ref[g, :] += cnt[0]

    @pl.when(i == (N // BT) - 1)
    def _finish():
        probs_ref[...] = probs_ref[...] * (1.0 / N)


def kernel(x, W, b, codebook):
    x2 = x.reshape(N, D)
    b2 = b.reshape(1, G * V)
    cb = codebook.reshape(G * V, DG).astype(jnp.bfloat16)
    out, probs = pl.pallas_call(
        _vq_kernel,
        grid=(N // BT,),
        in_specs=[
            pl.BlockSpec((BT, D), lambda i: (i, 0)),
            pl.BlockSpec((D, G * V), lambda i: (0, 0)),
            pl.BlockSpec((1, G * V), lambda i: (0, 0)),
            pl.BlockSpec((G * V, DG), lambda i: (0, 0)),
        ],
        out_specs=[
            pl.BlockSpec((BT, D), lambda i: (i, 0)),
            pl.BlockSpec((G, V), lambda i: (0, 0)),
        ],
        out_shape=[
            jax.ShapeDtypeStruct((N, D), jnp.float32),
            jax.ShapeDtypeStruct((G, V), jnp.float32),
        ],
    )(x2, W, b2, cb)
    return out.reshape(B, T, D), probs


# speedup vs baseline: 1.0601x; 1.0601x over previous
import jax
import jax.numpy as jnp
from jax.experimental import pallas as pl

B, T, D = 16, 2048, 512
G, V = 2, 1024
DG = D // G
N = B * T
BT = 2048


def _vq_kernel(x_ref, w_ref, b_ref, cb_ref, out_ref, probs_ref):
    i = pl.program_id(0)

    @pl.when(i == 0)
    def _init():
        probs_ref[...] = jnp.zeros_like(probs_ref)

    logits = jnp.dot(x_ref[...], w_ref[...], preferred_element_type=jnp.float32)
    logits = logits + b_ref[...]
    for g in range(G):
        lg = logits[:, g * V:(g + 1) * V]
        m = jnp.max(lg, axis=1, keepdims=True)
        oh = (lg == m).astype(jnp.float32)
        out_ref[:, g * DG:(g + 1) * DG] = jnp.dot(
            oh, cb_ref[g * V:(g + 1) * V, :], preferred_element_type=jnp.float32)
        probs_ref[g, :] += jnp.sum(oh, axis=0)

    @pl.when(i == (N // BT) - 1)
    def _finish():
        probs_ref[...] = probs_ref[...] * (1.0 / N)


def kernel(x, W, b, codebook):
    x2 = x.reshape(N, D)
    b2 = b.reshape(1, G * V)
    cb = codebook.reshape(G * V, DG)
    out, probs = pl.pallas_call(
        _vq_kernel,
        grid=(N // BT,),
        in_specs=[
            pl.BlockSpec((BT, D), lambda i: (i, 0)),
            pl.BlockSpec((D, G * V), lambda i: (0, 0)),
            pl.BlockSpec((1, G * V), lambda i: (0, 0)),
            pl.BlockSpec((G * V, DG), lambda i: (0, 0)),
        ],
        out_specs=[
            pl.BlockSpec((BT, D), lambda i: (i, 0)),
            pl.BlockSpec((G, V), lambda i: (0, 0)),
        ],
        out_shape=[
            jax.ShapeDtypeStruct((N, D), jnp.float32),
            jax.ShapeDtypeStruct((G, V), jnp.float32),
        ],
    )(x2, W, b2, cb)
    return out.reshape(B, T, D), probs


# speedup vs baseline: 1.0670x; 1.0065x over previous
import jax
import jax.numpy as jnp
from jax.experimental import pallas as pl

B, T, D = 16, 2048, 512
G, V = 2, 1024
DG = D // G
N = B * T
BT = 1024


def _vq_kernel(x_ref, w_ref, b_ref, cb_ref, out_ref, probs_ref):
    i = pl.program_id(0)

    @pl.when(i == 0)
    def _init():
        probs_ref[...] = jnp.zeros_like(probs_ref)

    logits = jnp.dot(x_ref[...], w_ref[...], preferred_element_type=jnp.float32)
    logits = logits + b_ref[...]
    for g in range(G):
        lg = logits[:, g * V:(g + 1) * V]
        m = jnp.max(lg, axis=1, keepdims=True)
        oh = (lg == m).astype(jnp.float32)
        out_ref[:, g * DG:(g + 1) * DG] = jnp.dot(
            oh, cb_ref[g * V:(g + 1) * V, :], preferred_element_type=jnp.float32)
        probs_ref[g, :] += jnp.sum(oh, axis=0)

    @pl.when(i == (N // BT) - 1)
    def _finish():
        probs_ref[...] = probs_ref[...] * (1.0 / N)


def kernel(x, W, b, codebook):
    x2 = x.reshape(N, D)
    b2 = b.reshape(1, G * V)
    cb = codebook.reshape(G * V, DG)
    out, probs = pl.pallas_call(
        _vq_kernel,
        grid=(N // BT,),
        in_specs=[
            pl.BlockSpec((BT, D), lambda i: (i, 0)),
            pl.BlockSpec((D, G * V), lambda i: (0, 0)),
            pl.BlockSpec((1, G * V), lambda i: (0, 0)),
            pl.BlockSpec((G * V, DG), lambda i: (0, 0)),
        ],
        out_specs=[
            pl.BlockSpec((BT, D), lambda i: (i, 0)),
            pl.BlockSpec((G, V), lambda i: (0, 0)),
        ],
        out_shape=[
            jax.ShapeDtypeStruct((N, D), jnp.float32),
            jax.ShapeDtypeStruct((G, V), jnp.float32),
        ],
    )(x2, W, b2, cb)
    return out.reshape(B, T, D), probs
